# baseline (speedup 1.0000x reference)
import functools
import jax
import jax.numpy as jnp
from jax import lax
from jax.experimental import pallas as pl
from jax.experimental.pallas import tpu as pltpu
from jax.experimental.pallas import tpu_sc as plsc

N_BINS_K = 15
ROW_BLOCK = 50000
NSPLIT = 2
NSEG = 3000
HOFF = 3008
HSTRIDE = 6017
NW = 32
CHUNK = 2000


def _dense_body(nblocks, total_rows):
    def body(*refs):
        logit_refs = refs[:NSPLIT]
        label_refs = refs[NSPLIT:2 * NSPLIT]
        uppers_ref = refs[2 * NSPLIT]
        conf_refs = refs[2 * NSPLIT + 1:2 * NSPLIT + 1 + NSPLIT]
        seg_refs = refs[2 * NSPLIT + 1 + NSPLIT:]
        i = pl.program_id(0)
        uppers = uppers_ref[...][0]

        for j in range(NSPLIT):
            x = logit_refs[j][...]
            b, c = x.shape
            xt = x.T
            m = jnp.max(xt, axis=0, keepdims=True)
            s = jnp.sum(jnp.exp(xt - m), axis=0, keepdims=True)
            conf = 1.0 / s
            cls_iota = jax.lax.broadcasted_iota(jnp.int32, (c, b), 0)
            cand = jnp.where(xt == m, cls_iota, c)
            pred = jnp.min(cand, axis=0, keepdims=True)
            labels = label_refs[j][...][:, 0, :]
            acc_i = (pred == labels).astype(jnp.int32)
            bin_idx = jnp.minimum(
                jnp.sum((uppers < conf).astype(jnp.int32), axis=0,
                        keepdims=True),
                N_BINS_K - 1)
            seg2 = pred * N_BINS_K + bin_idx + (N_BINS_K * 100) * acc_i
            if nblocks * b * NSPLIT != total_rows:
                col = ((i * NSPLIT + j) * b
                       + jax.lax.broadcasted_iota(jnp.int32, (1, b), 1))
                seg2 = jnp.where(col < total_rows, seg2, NSEG)
            conf_refs[j][...] = conf[None]
            seg_refs[j][...] = seg2[None]
    return body


def _sc_hist_body(nchunks_half):
    def body(seg0_hbm, conf0_hbm, seg1_hbm, conf1_hbm, out_hbm,
             seg_v, conf_v, hist_v, out_v):
        cid = lax.axis_index("c")
        sid = lax.axis_index("s")
        wid = sid * 2 + cid
        lane_base = lax.iota(jnp.int32, 16) * HSTRIDE
        zv = jnp.zeros((16,), jnp.float32)
        ones = jnp.ones((16,), jnp.float32)

        def zero_body(i, _):
            hist_v[pl.ds(i * 16, 16)] = zv
            return 0
        lax.fori_loop(0, HSTRIDE, zero_body, 0)

        for seg_hbm, conf_hbm in ((seg0_hbm, conf0_hbm),
                                  (seg1_hbm, conf1_hbm)):
            def chunk_body(t, _, seg_hbm=seg_hbm, conf_hbm=conf_hbm):
                k = t * NW + wid

                @pl.when(k < nchunks_half)
                def _():
                    base = k * CHUNK
                    pltpu.sync_copy(seg_hbm.at[pl.ds(base, CHUNK)], seg_v)
                    pltpu.sync_copy(conf_hbm.at[pl.ds(base, CHUNK)], conf_v)

                    def inner(j, _):
                        sv = seg_v[pl.ds(j * 16, 16)]
                        cv = conf_v[pl.ds(j * 16, 16)]
                        idx = sv + lane_base
                        plsc.addupdate_scatter(hist_v, [idx], ones)
                        plsc.addupdate_scatter(hist_v, [idx + HOFF], cv)
                        return 0
                    lax.fori_loop(0, CHUNK // 16, inner, 0)
                return 0
            lax.fori_loop(0, (nchunks_half + NW - 1) // NW, chunk_body, 0)

        def merge_body(j, _):
            acc = hist_v[pl.ds(j * 16, 16)]
            for l in range(1, 16):
                acc = acc + hist_v[pl.ds(l * HSTRIDE + j * 16, 16)]
            out_v[pl.ds(j * 16, 16)] = acc
            return 0
        lax.fori_loop(0, 376, merge_body, 0)
        pltpu.sync_copy(out_v, out_hbm.at[wid])
    return body


def _final_body(parts_ref, out_ref):
    p = parts_ref[...]
    h = jnp.sum(p, axis=0, keepdims=True)
    l2 = N_BINS_K * 100
    count = h[:, 0:l2] + h[:, l2:2 * l2]
    acc_sum = h[:, l2:2 * l2]
    conf_sum = h[:, HOFF:HOFF + l2] + h[:, HOFF + l2:HOFF + 2 * l2]
    jj = jax.lax.broadcasted_iota(jnp.int32, (l2, 100), 0)
    cc = jax.lax.broadcasted_iota(jnp.int32, (l2, 100), 1)
    sel = jnp.where(jj // N_BINS_K == cc, 1.0, 0.0)
    class_count = jax.lax.dot_general(
        count, sel, (((1,), (0,)), ((), ())),
        preferred_element_type=jnp.float32)
    ccnt_b = jax.lax.dot_general(
        jnp.maximum(class_count, 1.0), sel, (((1,), (1,)), ((), ())),
        preferred_element_type=jnp.float32)
    safe = jnp.maximum(count, 1.0)
    gap = jnp.where(count > 0.0,
                    jnp.abs(conf_sum / safe - acc_sum / safe) * count / ccnt_b,
                    0.0)
    out_ref[...] = jax.lax.dot_general(
        gap, sel, (((1,), (0,)), ((), ())),
        preferred_element_type=jnp.float32)


def kernel(logits, labels):
    n, c = logits.shape
    labels = labels.astype(jnp.int32)

    bs = ROW_BLOCK // NSPLIT
    nblocks = -(-n // ROW_BLOCK)
    npad = nblocks * ROW_BLOCK
    if npad != n:
        logits = jnp.pad(logits, ((0, npad - n), (0, 0)))
        labels = jnp.pad(labels, (0, npad - n))
    labels3 = labels.reshape(nblocks * NSPLIT, 1, bs)
    uppers = jnp.linspace(0.0, 1.0, N_BINS_K + 1)[1:].astype(
        jnp.float32).reshape(1, N_BINS_K, 1)

    logit_specs = [
        pl.BlockSpec((bs, c), lambda i, j=j: (NSPLIT * i + j, 0))
        for j in range(NSPLIT)
    ]
    label_specs = [
        pl.BlockSpec((1, 1, bs), lambda i, j=j: (NSPLIT * i + j, 0, 0))
        for j in range(NSPLIT)
    ]
    out_specs = [
        pl.BlockSpec((1, 1, bs), lambda i: (i, 0, 0))
        for _ in range(NSPLIT)
    ]

    dense_out = pl.pallas_call(
        _dense_body(nblocks, n),
        grid=(nblocks,),
        in_specs=logit_specs + label_specs + [
            pl.BlockSpec((1, N_BINS_K, 1), lambda i: (0, 0, 0)),
        ],
        out_specs=out_specs + out_specs,
        out_shape=(
            [jax.ShapeDtypeStruct((nblocks, 1, bs), jnp.float32)
             for _ in range(NSPLIT)]
            + [jax.ShapeDtypeStruct((nblocks, 1, bs), jnp.int32)
               for _ in range(NSPLIT)]
        ),
    )(*([logits] * NSPLIT), *([labels3] * NSPLIT), uppers)
    conf_parts = [a.reshape(-1) for a in dense_out[:NSPLIT]]
    seg_parts = [a.reshape(-1) for a in dense_out[NSPLIT:]]

    nchunks_half = (npad // NSPLIT) // CHUNK
    mesh = plsc.VectorSubcoreMesh(core_axis_name="c", subcore_axis_name="s")
    sc_hist = functools.partial(
        pl.kernel,
        mesh=mesh,
        compiler_params=pltpu.CompilerParams(needs_layout_passes=False),
        out_type=jax.ShapeDtypeStruct((NW, 6016), jnp.float32),
        scratch_types=[
            pltpu.VMEM((CHUNK,), jnp.int32),
            pltpu.VMEM((CHUNK,), jnp.float32),
            pltpu.VMEM((16 * HSTRIDE,), jnp.float32),
            pltpu.VMEM((6016,), jnp.float32),
        ],
    )(_sc_hist_body(nchunks_half))
    parts = sc_hist(seg_parts[0], conf_parts[0], seg_parts[1], conf_parts[1])

    out = pl.pallas_call(
        _final_body,
        out_shape=jax.ShapeDtypeStruct((1, c), jnp.float32),
    )(parts)
    return out.reshape(c)


# speedup vs baseline: 1.0038x; 1.0038x over previous
import functools
import jax
import jax.numpy as jnp
from jax import lax
from jax.experimental import pallas as pl
from jax.experimental.pallas import tpu as pltpu
from jax.experimental.pallas import tpu_sc as plsc

N_BINS_K = 15
ROW_BLOCK = 50000
NSPLIT = 2
NSEG = 3000
HOFF = 3008
HSTRIDE = 6017
NW = 32
CHUNK = 5000


def _dense_body(nblocks, total_rows):
    def body(*refs):
        logit_refs = refs[:NSPLIT]
        label_refs = refs[NSPLIT:2 * NSPLIT]
        uppers_ref = refs[2 * NSPLIT]
        conf_refs = refs[2 * NSPLIT + 1:2 * NSPLIT + 1 + NSPLIT]
        seg_refs = refs[2 * NSPLIT + 1 + NSPLIT:]
        i = pl.program_id(0)
        uppers = uppers_ref[...][0]

        for j in range(NSPLIT):
            x = logit_refs[j][...]
            b, c = x.shape
            xt = x.T
            m = jnp.max(xt, axis=0, keepdims=True)
            s = jnp.sum(jnp.exp(xt - m), axis=0, keepdims=True)
            conf = 1.0 / s
            cls_iota = jax.lax.broadcasted_iota(jnp.int32, (c, b), 0)
            cand = jnp.where(xt == m, cls_iota, c)
            pred = jnp.min(cand, axis=0, keepdims=True)
            labels = label_refs[j][...][:, 0, :]
            acc_i = (pred == labels).astype(jnp.int32)
            bin_idx = jnp.minimum(
                jnp.sum((uppers < conf).astype(jnp.int32), axis=0,
                        keepdims=True),
                N_BINS_K - 1)
            seg2 = pred * N_BINS_K + bin_idx + (N_BINS_K * 100) * acc_i
            if nblocks * b * NSPLIT != total_rows:
                col = ((i * NSPLIT + j) * b
                       + jax.lax.broadcasted_iota(jnp.int32, (1, b), 1))
                seg2 = jnp.where(col < total_rows, seg2, NSEG)
            conf_refs[j][...] = conf[None]
            seg_refs[j][...] = seg2[None]
    return body


def _sc_hist_body(nchunks_half):
    def body(seg0_hbm, conf0_hbm, seg1_hbm, conf1_hbm, out_hbm,
             seg_v, conf_v, hist_v, out_v):
        cid = lax.axis_index("c")
        sid = lax.axis_index("s")
        wid = sid * 2 + cid
        lane_base = lax.iota(jnp.int32, 16) * HSTRIDE
        zv = jnp.zeros((16,), jnp.float32)
        ones = jnp.ones((16,), jnp.float32)

        def zero_body(i, _):
            hist_v[pl.ds(i * 16, 16)] = zv
            return 0
        lax.fori_loop(0, HSTRIDE, zero_body, 0)

        for seg_hbm, conf_hbm in ((seg0_hbm, conf0_hbm),
                                  (seg1_hbm, conf1_hbm)):
            def chunk_body(t, _, seg_hbm=seg_hbm, conf_hbm=conf_hbm):
                k = t * NW + wid

                @pl.when(k < nchunks_half)
                def _():
                    base = k * CHUNK
                    pltpu.sync_copy(seg_hbm.at[pl.ds(base, CHUNK)], seg_v)
                    pltpu.sync_copy(conf_hbm.at[pl.ds(base, CHUNK)], conf_v)

                    def inner(j, _):
                        sv = seg_v[pl.ds(j * 16, 16)]
                        cv = conf_v[pl.ds(j * 16, 16)]
                        idx = sv + lane_base
                        plsc.addupdate_scatter(hist_v, [idx], ones)
                        plsc.addupdate_scatter(hist_v, [idx + HOFF], cv)
                        return 0
                    lax.fori_loop(0, CHUNK // 16, inner, 0)
                return 0
            lax.fori_loop(0, (nchunks_half + NW - 1) // NW, chunk_body, 0)

        def merge_body(j, _):
            acc = hist_v[pl.ds(j * 16, 16)]
            for l in range(1, 16):
                acc = acc + hist_v[pl.ds(l * HSTRIDE + j * 16, 16)]
            out_v[pl.ds(j * 16, 16)] = acc
            return 0
        lax.fori_loop(0, 376, merge_body, 0)
        pltpu.sync_copy(out_v, out_hbm.at[wid])
    return body


def _final_body(parts_ref, out_ref):
    p = parts_ref[...]
    h = jnp.sum(p, axis=0, keepdims=True)
    l2 = N_BINS_K * 100
    count = h[:, 0:l2] + h[:, l2:2 * l2]
    acc_sum = h[:, l2:2 * l2]
    conf_sum = h[:, HOFF:HOFF + l2] + h[:, HOFF + l2:HOFF + 2 * l2]
    jj = jax.lax.broadcasted_iota(jnp.int32, (l2, 100), 0)
    cc = jax.lax.broadcasted_iota(jnp.int32, (l2, 100), 1)
    sel = jnp.where(jj // N_BINS_K == cc, 1.0, 0.0)
    class_count = jax.lax.dot_general(
        count, sel, (((1,), (0,)), ((), ())),
        preferred_element_type=jnp.float32)
    ccnt_b = jax.lax.dot_general(
        jnp.maximum(class_count, 1.0), sel, (((1,), (1,)), ((), ())),
        preferred_element_type=jnp.float32)
    safe = jnp.maximum(count, 1.0)
    gap = jnp.where(count > 0.0,
                    jnp.abs(conf_sum / safe - acc_sum / safe) * count / ccnt_b,
                    0.0)
    out_ref[...] = jax.lax.dot_general(
        gap, sel, (((1,), (0,)), ((), ())),
        preferred_element_type=jnp.float32)


def kernel(logits, labels):
    n, c = logits.shape
    labels = labels.astype(jnp.int32)

    bs = ROW_BLOCK // NSPLIT
    nblocks = -(-n // ROW_BLOCK)
    npad = nblocks * ROW_BLOCK
    if npad != n:
        logits = jnp.pad(logits, ((0, npad - n), (0, 0)))
        labels = jnp.pad(labels, (0, npad - n))
    labels3 = labels.reshape(nblocks * NSPLIT, 1, bs)
    uppers = jnp.linspace(0.0, 1.0, N_BINS_K + 1)[1:].astype(
        jnp.float32).reshape(1, N_BINS_K, 1)

    logit_specs = [
        pl.BlockSpec((bs, c), lambda i, j=j: (NSPLIT * i + j, 0))
        for j in range(NSPLIT)
    ]
    label_specs = [
        pl.BlockSpec((1, 1, bs), lambda i, j=j: (NSPLIT * i + j, 0, 0))
        for j in range(NSPLIT)
    ]
    out_specs = [
        pl.BlockSpec((1, 1, bs), lambda i: (i, 0, 0))
        for _ in range(NSPLIT)
    ]

    dense_out = pl.pallas_call(
        _dense_body(nblocks, n),
        grid=(nblocks,),
        in_specs=logit_specs + label_specs + [
            pl.BlockSpec((1, N_BINS_K, 1), lambda i: (0, 0, 0)),
        ],
        out_specs=out_specs + out_specs,
        out_shape=(
            [jax.ShapeDtypeStruct((nblocks, 1, bs), jnp.float32)
             for _ in range(NSPLIT)]
            + [jax.ShapeDtypeStruct((nblocks, 1, bs), jnp.int32)
               for _ in range(NSPLIT)]
        ),
    )(*([logits] * NSPLIT), *([labels3] * NSPLIT), uppers)
    conf_parts = [a.reshape(-1) for a in dense_out[:NSPLIT]]
    seg_parts = [a.reshape(-1) for a in dense_out[NSPLIT:]]

    nchunks_half = (npad // NSPLIT) // CHUNK
    mesh = plsc.VectorSubcoreMesh(core_axis_name="c", subcore_axis_name="s")
    sc_hist = functools.partial(
        pl.kernel,
        mesh=mesh,
        compiler_params=pltpu.CompilerParams(needs_layout_passes=False),
        out_type=jax.ShapeDtypeStruct((NW, 6016), jnp.float32),
        scratch_types=[
            pltpu.VMEM((CHUNK,), jnp.int32),
            pltpu.VMEM((CHUNK,), jnp.float32),
            pltpu.VMEM((16 * HSTRIDE,), jnp.float32),
            pltpu.VMEM((6016,), jnp.float32),
        ],
    )(_sc_hist_body(nchunks_half))
    parts = sc_hist(seg_parts[0], conf_parts[0], seg_parts[1], conf_parts[1])

    out = pl.pallas_call(
        _final_body,
        out_shape=jax.ShapeDtypeStruct((1, c), jnp.float32),
    )(parts)
    return out.reshape(c)
